# baseline (speedup 1.0000x reference)
import functools

import jax
import jax.numpy as jnp
from jax import lax
from jax.experimental import pallas as pl
from jax.experimental.pallas import tpu as pltpu
from jax.experimental.pallas import tpu_sc as plsc

_VOCAB = 100000
_D = 128
_H = 512
_B = 16384

_NC, _NS = 2, 16
_NW = _NC * _NS
_BPW = _B // _NW
_CH = 4
_CROWS = _BPW // _CH


def _sc_gather(tab_hbm, ids_hbm, out_hbm, idx_v, rows0, rows1, sem0, sem1):
    wid = lax.axis_index("s") * _NC + lax.axis_index("c")
    base = wid * _BPW
    pltpu.sync_copy(ids_hbm.at[pl.ds(base, _BPW)], idx_v)
    bufs = (rows0, rows1)
    sems = (sem0, sem1)

    def _start(c, buf, sem):
        pltpu.async_copy(tab_hbm.at[idx_v.at[pl.ds(c * _CROWS, _CROWS)]],
                         buf, sem)

    _start(0, bufs[0], sems[0])
    for c in range(_CH):
        if c + 1 < _CH:
            _start(c + 1, bufs[(c + 1) % 2], sems[(c + 1) % 2])
        buf = bufs[c % 2]
        pltpu.make_async_copy(tab_hbm.at[idx_v.at[pl.ds(c * _CROWS, _CROWS)]],
                              buf, sems[c % 2]).wait()
        pltpu.sync_copy(buf, out_hbm.at[pl.ds(base + c * _CROWS, _CROWS)])


@functools.lru_cache(maxsize=None)
def _gather_call():
    return pl.kernel(
        _sc_gather,
        mesh=plsc.VectorSubcoreMesh(core_axis_name="c", subcore_axis_name="s"),
        out_type=jax.ShapeDtypeStruct((_B, _D), jnp.float32),
        scratch_types=[
            pltpu.VMEM((_BPW,), jnp.int32),
            pltpu.VMEM((_CROWS, _D), jnp.float32),
            pltpu.VMEM((_CROWS, _D), jnp.float32),
            pltpu.SemaphoreType.DMA,
            pltpu.SemaphoreType.DMA,
        ],
    )


_BLK = 1024
_C1 = 0.7978845608028654
_C2 = _C1 * 0.044715


def _lean_gelu(x):
    t = jnp.tanh(x * (_C1 + _C2 * (x * x)))
    hx = 0.5 * x
    return hx + hx * t


def _mlp_body(x_ref, w1_ref, b1_ref, w2_ref, b2_ref, o_ref):
    h = jnp.dot(x_ref[...], w1_ref[...], preferred_element_type=jnp.float32)
    h = _lean_gelu(h + b1_ref[...])
    o_ref[...] = (
        jnp.dot(h, w2_ref[...], preferred_element_type=jnp.float32)
        + b2_ref[...])


def _mlp(x, W1, b1, W2, b2):
    row_spec = pl.BlockSpec((_BLK, _D), lambda i: (i, 0))
    return pl.pallas_call(
        _mlp_body,
        grid=(_B // _BLK,),
        in_specs=[
            row_spec,
            pl.BlockSpec((_D, _H), lambda i: (0, 0)),
            pl.BlockSpec((1, _H), lambda i: (0, 0)),
            pl.BlockSpec((_H, _D), lambda i: (0, 0)),
            pl.BlockSpec((1, _D), lambda i: (0, 0)),
        ],
        out_specs=row_spec,
        out_shape=jax.ShapeDtypeStruct((_B, _D), jnp.float32),
    )(x, W1, b1, W2, b2)


@jax.jit
def kernel(query_ids, entity_ids, query_emb, ent_emb, W1, b1, W2, b2):
    b1r = b1.reshape(1, _H)
    b2r = b2.reshape(1, _D)
    gather = _gather_call()
    q_rows = gather(query_emb, query_ids.astype(jnp.int32))
    e_rows = gather(ent_emb, entity_ids.astype(jnp.int32))
    q_out = _mlp(q_rows, W1, b1r, W2, b2r)
    e_out = _mlp(e_rows, W1, b1r, W2, b2r)
    return q_out, e_out


# speedup vs baseline: 1.0984x; 1.0984x over previous
import functools

import jax
import jax.numpy as jnp
from jax import lax
from jax.experimental import pallas as pl
from jax.experimental.pallas import tpu as pltpu
from jax.experimental.pallas import tpu_sc as plsc

_VOCAB = 100000
_D = 128
_H = 512
_B = 16384

_NC, _NS = 2, 16
_NW = _NC * _NS
_BPW = _B // _NW
_CH = 4
_CROWS = _BPW // _CH


def _sc_gather(qtab_hbm, etab_hbm, qids_hbm, eids_hbm, q_out, e_out,
               qidx_v, eidx_v, rows0, rows1, sem0, sem1):
    wid = lax.axis_index("s") * _NC + lax.axis_index("c")
    base = wid * _BPW
    pltpu.sync_copy(qids_hbm.at[pl.ds(base, _BPW)], qidx_v)
    pltpu.sync_copy(eids_hbm.at[pl.ds(base, _BPW)], eidx_v)
    bufs = (rows0, rows1)
    sems = (sem0, sem1)
    work = [(qtab_hbm, qidx_v, q_out, c) for c in range(_CH)]
    work += [(etab_hbm, eidx_v, e_out, c) for c in range(_CH)]

    def _start(j, buf, sem):
        tab, idx, _, c = work[j]
        pltpu.async_copy(tab.at[idx.at[pl.ds(c * _CROWS, _CROWS)]], buf, sem)

    _start(0, bufs[0], sems[0])
    for j in range(len(work)):
        if j + 1 < len(work):
            _start(j + 1, bufs[(j + 1) % 2], sems[(j + 1) % 2])
        tab, idx, out, c = work[j]
        buf = bufs[j % 2]
        pltpu.make_async_copy(tab.at[idx.at[pl.ds(c * _CROWS, _CROWS)]],
                              buf, sems[j % 2]).wait()
        pltpu.sync_copy(buf, out.at[pl.ds(base + c * _CROWS, _CROWS)])


@functools.lru_cache(maxsize=None)
def _gather_call():
    return pl.kernel(
        _sc_gather,
        mesh=plsc.VectorSubcoreMesh(core_axis_name="c", subcore_axis_name="s"),
        out_type=[
            jax.ShapeDtypeStruct((_B, _D), jnp.float32),
            jax.ShapeDtypeStruct((_B, _D), jnp.float32),
        ],
        scratch_types=[
            pltpu.VMEM((_BPW,), jnp.int32),
            pltpu.VMEM((_BPW,), jnp.int32),
            pltpu.VMEM((_CROWS, _D), jnp.float32),
            pltpu.VMEM((_CROWS, _D), jnp.float32),
            pltpu.SemaphoreType.DMA,
            pltpu.SemaphoreType.DMA,
        ],
    )


_BLK = 1024
_C1 = 0.7978845608028654
_C2 = _C1 * 0.044715


def _lean_gelu(x):
    t = jnp.tanh(x * (_C1 + _C2 * (x * x)))
    hx = 0.5 * x
    return hx + hx * t


def _mlp_body(xq_ref, xe_ref, w1_ref, b1_ref, w2_ref, b2_ref,
              oq_ref, oe_ref):
    w1 = w1_ref[...]
    w2 = w2_ref[...]
    b1 = b1_ref[...]
    b2 = b2_ref[...]
    hq = _lean_gelu(
        jnp.dot(xq_ref[...], w1, preferred_element_type=jnp.float32) + b1)
    oq_ref[...] = jnp.dot(hq, w2, preferred_element_type=jnp.float32) + b2
    he = _lean_gelu(
        jnp.dot(xe_ref[...], w1, preferred_element_type=jnp.float32) + b1)
    oe_ref[...] = jnp.dot(he, w2, preferred_element_type=jnp.float32) + b2


def _mlp2(xq, xe, W1, b1, W2, b2):
    row_spec = pl.BlockSpec((_BLK, _D), lambda i: (i, 0))
    return pl.pallas_call(
        _mlp_body,
        grid=(_B // _BLK,),
        in_specs=[
            row_spec,
            row_spec,
            pl.BlockSpec((_D, _H), lambda i: (0, 0)),
            pl.BlockSpec((1, _H), lambda i: (0, 0)),
            pl.BlockSpec((_H, _D), lambda i: (0, 0)),
            pl.BlockSpec((1, _D), lambda i: (0, 0)),
        ],
        out_specs=[row_spec, row_spec],
        out_shape=[
            jax.ShapeDtypeStruct((_B, _D), jnp.float32),
            jax.ShapeDtypeStruct((_B, _D), jnp.float32),
        ],
    )(xq, xe, W1, b1, W2, b2)


@jax.jit
def kernel(query_ids, entity_ids, query_emb, ent_emb, W1, b1, W2, b2):
    q_rows, e_rows = _gather_call()(
        query_emb, ent_emb,
        query_ids.astype(jnp.int32), entity_ids.astype(jnp.int32))
    return _mlp2(q_rows, e_rows, W1, b1.reshape(1, _H), W2,
                 b2.reshape(1, _D))


# speedup vs baseline: 1.1172x; 1.0171x over previous
import functools

import jax
import jax.numpy as jnp
from jax import lax
from jax.experimental import pallas as pl
from jax.experimental.pallas import tpu as pltpu
from jax.experimental.pallas import tpu_sc as plsc

_VOCAB = 100000
_D = 128
_H = 512
_B = 16384

_NC, _NS = 2, 16
_NW = _NC * _NS
_BPW = _B // _NW
_CH = 4
_CROWS = _BPW // _CH


def _sc_gather(qtab_hbm, etab_hbm, qids_hbm, eids_hbm, q_out, e_out,
               qidx_v, eidx_v, rows0, rows1, sem0, sem1):
    wid = lax.axis_index("s") * _NC + lax.axis_index("c")
    base = wid * _BPW
    pltpu.sync_copy(qids_hbm.at[pl.ds(base, _BPW)], qidx_v)
    pltpu.sync_copy(eids_hbm.at[pl.ds(base, _BPW)], eidx_v)
    bufs = (rows0, rows1)
    sems = (sem0, sem1)
    work = [(qtab_hbm, qidx_v, q_out, c) for c in range(_CH)]
    work += [(etab_hbm, eidx_v, e_out, c) for c in range(_CH)]

    def _start(j, buf, sem):
        tab, idx, _, c = work[j]
        pltpu.async_copy(tab.at[idx.at[pl.ds(c * _CROWS, _CROWS)]], buf, sem)

    _start(0, bufs[0], sems[0])
    for j in range(len(work)):
        if j + 1 < len(work):
            _start(j + 1, bufs[(j + 1) % 2], sems[(j + 1) % 2])
        tab, idx, out, c = work[j]
        buf = bufs[j % 2]
        pltpu.make_async_copy(tab.at[idx.at[pl.ds(c * _CROWS, _CROWS)]],
                              buf, sems[j % 2]).wait()
        pltpu.sync_copy(buf, out.at[pl.ds(base + c * _CROWS, _CROWS)])


@functools.lru_cache(maxsize=None)
def _gather_call():
    return pl.kernel(
        _sc_gather,
        mesh=plsc.VectorSubcoreMesh(core_axis_name="c", subcore_axis_name="s"),
        out_type=[
            jax.ShapeDtypeStruct((_B, _D), jnp.float32),
            jax.ShapeDtypeStruct((_B, _D), jnp.float32),
        ],
        scratch_types=[
            pltpu.VMEM((_BPW,), jnp.int32),
            pltpu.VMEM((_BPW,), jnp.int32),
            pltpu.VMEM((_CROWS, _D), jnp.float32),
            pltpu.VMEM((_CROWS, _D), jnp.float32),
            pltpu.SemaphoreType.DMA,
            pltpu.SemaphoreType.DMA,
        ],
    )


_BLK = 1024
_C1 = 0.7978845608028654
_C2 = _C1 * 0.044715


def _gelu2(x):
    t = jnp.tanh(x * (_C1 + _C2 * (x * x)))
    return x + x * t


def _mlp_body(xq_ref, xe_ref, w1_ref, b1_ref, w2_ref, b2_ref,
              oq_ref, oe_ref):
    w1 = w1_ref[...]
    w2 = w2_ref[...]
    b1 = b1_ref[...]
    b2 = b2_ref[...]
    hq = _gelu2(
        jnp.dot(xq_ref[...].astype(jnp.bfloat16), w1,
                preferred_element_type=jnp.float32) + b1)
    oq_ref[...] = jnp.dot(hq.astype(jnp.bfloat16), w2,
                          preferred_element_type=jnp.float32) + b2
    he = _gelu2(
        jnp.dot(xe_ref[...].astype(jnp.bfloat16), w1,
                preferred_element_type=jnp.float32) + b1)
    oe_ref[...] = jnp.dot(he.astype(jnp.bfloat16), w2,
                          preferred_element_type=jnp.float32) + b2


def _mlp2(xq, xe, W1, b1, W2, b2):
    row_spec = pl.BlockSpec((_BLK, _D), lambda i: (i, 0))
    return pl.pallas_call(
        _mlp_body,
        grid=(_B // _BLK,),
        in_specs=[
            row_spec,
            row_spec,
            pl.BlockSpec((_D, _H), lambda i: (0, 0)),
            pl.BlockSpec((1, _H), lambda i: (0, 0)),
            pl.BlockSpec((_H, _D), lambda i: (0, 0)),
            pl.BlockSpec((1, _D), lambda i: (0, 0)),
        ],
        out_specs=[row_spec, row_spec],
        out_shape=[
            jax.ShapeDtypeStruct((_B, _D), jnp.float32),
            jax.ShapeDtypeStruct((_B, _D), jnp.float32),
        ],
    )(xq, xe, W1, b1, W2, b2)


@jax.jit
def kernel(query_ids, entity_ids, query_emb, ent_emb, W1, b1, W2, b2):
    q_rows, e_rows = _gather_call()(
        query_emb, ent_emb,
        query_ids.astype(jnp.int32), entity_ids.astype(jnp.int32))
    return _mlp2(q_rows, e_rows,
                 W1.astype(jnp.bfloat16), b1.reshape(1, _H),
                 (0.5 * W2).astype(jnp.bfloat16), b2.reshape(1, _D))


# speedup vs baseline: 1.1301x; 1.0116x over previous
import functools

import jax
import jax.numpy as jnp
from jax import lax
from jax.experimental import pallas as pl
from jax.experimental.pallas import tpu as pltpu
from jax.experimental.pallas import tpu_sc as plsc

_VOCAB = 100000
_D = 128
_H = 512
_B = 16384

_NCHUNK = 2
_CB = _B // _NCHUNK

_NC, _NS = 2, 16
_NW = _NC * _NS
_BPW = _CB // _NW
_CROWS = 128
_CH = _BPW // _CROWS


def _sc_gather(qtab_hbm, etab_hbm, qids_hbm, eids_hbm, q_out, e_out,
               qidx_v, eidx_v, rows0, rows1, sem0, sem1):
    wid = lax.axis_index("s") * _NC + lax.axis_index("c")
    base = wid * _BPW
    pltpu.sync_copy(qids_hbm.at[pl.ds(base, _BPW)], qidx_v)
    pltpu.sync_copy(eids_hbm.at[pl.ds(base, _BPW)], eidx_v)
    bufs = (rows0, rows1)
    sems = (sem0, sem1)
    work = [(qtab_hbm, qidx_v, q_out, c) for c in range(_CH)]
    work += [(etab_hbm, eidx_v, e_out, c) for c in range(_CH)]

    def _start(j, buf, sem):
        tab, idx, _, c = work[j]
        pltpu.async_copy(tab.at[idx.at[pl.ds(c * _CROWS, _CROWS)]], buf, sem)

    _start(0, bufs[0], sems[0])
    for j in range(len(work)):
        if j + 1 < len(work):
            _start(j + 1, bufs[(j + 1) % 2], sems[(j + 1) % 2])
        tab, idx, out, c = work[j]
        buf = bufs[j % 2]
        pltpu.make_async_copy(tab.at[idx.at[pl.ds(c * _CROWS, _CROWS)]],
                              buf, sems[j % 2]).wait()
        pltpu.sync_copy(buf, out.at[pl.ds(base + c * _CROWS, _CROWS)])


@functools.lru_cache(maxsize=None)
def _gather_call():
    return pl.kernel(
        _sc_gather,
        mesh=plsc.VectorSubcoreMesh(core_axis_name="c", subcore_axis_name="s"),
        out_type=[
            jax.ShapeDtypeStruct((_CB, _D), jnp.float32),
            jax.ShapeDtypeStruct((_CB, _D), jnp.float32),
        ],
        scratch_types=[
            pltpu.VMEM((_BPW,), jnp.int32),
            pltpu.VMEM((_BPW,), jnp.int32),
            pltpu.VMEM((_CROWS, _D), jnp.float32),
            pltpu.VMEM((_CROWS, _D), jnp.float32),
            pltpu.SemaphoreType.DMA,
            pltpu.SemaphoreType.DMA,
        ],
    )


_BLK = 1024
_GSTEPS = _CB // _BLK
_C1 = 0.7978845608028654
_C2 = _C1 * 0.044715


def _gelu2(x):
    t = jnp.tanh(x * (_C1 + _C2 * (x * x)))
    return x + x * t


def _mlp_body(xq_ref, xe_ref, w1_ref, b1_ref, w2_ref, b2_ref,
              oq_ref, oe_ref):
    w1 = w1_ref[...]
    w2 = w2_ref[...]
    b1 = b1_ref[...]
    b2 = b2_ref[...]
    hq = _gelu2(
        jnp.dot(xq_ref[...].astype(jnp.bfloat16), w1,
                preferred_element_type=jnp.float32) + b1)
    oq_ref[...] = jnp.dot(hq.astype(jnp.bfloat16), w2,
                          preferred_element_type=jnp.float32) + b2
    he = _gelu2(
        jnp.dot(xe_ref[...].astype(jnp.bfloat16), w1,
                preferred_element_type=jnp.float32) + b1)
    oe_ref[...] = jnp.dot(he.astype(jnp.bfloat16), w2,
                          preferred_element_type=jnp.float32) + b2


def _mlp_chunk0(body_args):
    row_in = pl.BlockSpec((_BLK, _D), lambda i: (i, 0))
    row_out = pl.BlockSpec((_BLK, _D), lambda i: (i, 0))
    return pl.pallas_call(
        _mlp_body,
        grid=(_GSTEPS,),
        in_specs=[
            row_in,
            row_in,
            pl.BlockSpec((_D, _H), lambda i: (0, 0)),
            pl.BlockSpec((1, _H), lambda i: (0, 0)),
            pl.BlockSpec((_H, _D), lambda i: (0, 0)),
            pl.BlockSpec((1, _D), lambda i: (0, 0)),
        ],
        out_specs=[row_out, row_out],
        out_shape=[
            jax.ShapeDtypeStruct((_B, _D), jnp.float32),
            jax.ShapeDtypeStruct((_B, _D), jnp.float32),
        ],
    )(*body_args)


def _mlp_chunk1_body(xq_ref, xe_ref, w1_ref, b1_ref, w2_ref, b2_ref,
                     _aq_ref, _ae_ref, oq_ref, oe_ref):
    _mlp_body(xq_ref, xe_ref, w1_ref, b1_ref, w2_ref, b2_ref,
              oq_ref, oe_ref)


def _mlp_chunk1(body_args, oq_prev, oe_prev):
    row_in = pl.BlockSpec((_BLK, _D), lambda i: (i, 0))
    row_out = pl.BlockSpec((_BLK, _D), lambda i: (i + _GSTEPS, 0))
    return pl.pallas_call(
        _mlp_chunk1_body,
        grid=(_GSTEPS,),
        in_specs=[
            row_in,
            row_in,
            pl.BlockSpec((_D, _H), lambda i: (0, 0)),
            pl.BlockSpec((1, _H), lambda i: (0, 0)),
            pl.BlockSpec((_H, _D), lambda i: (0, 0)),
            pl.BlockSpec((1, _D), lambda i: (0, 0)),
            pl.BlockSpec(memory_space=pl.ANY),
            pl.BlockSpec(memory_space=pl.ANY),
        ],
        out_specs=[row_out, row_out],
        out_shape=[
            jax.ShapeDtypeStruct((_B, _D), jnp.float32),
            jax.ShapeDtypeStruct((_B, _D), jnp.float32),
        ],
        input_output_aliases={6: 0, 7: 1},
    )(*body_args, oq_prev, oe_prev)


@jax.jit
def kernel(query_ids, entity_ids, query_emb, ent_emb, W1, b1, W2, b2):
    qids = query_ids.astype(jnp.int32)
    eids = entity_ids.astype(jnp.int32)
    w1 = W1.astype(jnp.bfloat16)
    w2 = (0.5 * W2).astype(jnp.bfloat16)
    b1r = b1.reshape(1, _H)
    b2r = b2.reshape(1, _D)
    gather = _gather_call()
    qr0, er0 = gather(query_emb, ent_emb, qids[:_CB], eids[:_CB])
    qr1, er1 = gather(query_emb, ent_emb, qids[_CB:], eids[_CB:])
    oq, oe = _mlp_chunk0((qr0, er0, w1, b1r, w2, b2r))
    return _mlp_chunk1((qr1, er1, w1, b1r, w2, b2r), oq, oe)


# speedup vs baseline: 1.2109x; 1.0715x over previous
import functools

import jax
import jax.numpy as jnp
from jax import lax
from jax.experimental import pallas as pl
from jax.experimental.pallas import tpu as pltpu
from jax.experimental.pallas import tpu_sc as plsc

_VOCAB = 100000
_D = 128
_H = 512
_B = 16384

_NCHUNK = 2
_CB = _B // _NCHUNK

_NC, _NS = 2, 16
_NW = _NC * _NS
_BPW = _CB // _NW
_CROWS = 128
_CH = _BPW // _CROWS


def _sc_gather(qtab_hbm, etab_hbm, qids_hbm, eids_hbm, q_out, e_out,
               qidx_v, eidx_v, rows0, rows1, sem0, sem1):
    wid = lax.axis_index("s") * _NC + lax.axis_index("c")
    base = wid * _BPW
    pltpu.sync_copy(qids_hbm.at[pl.ds(base, _BPW)], qidx_v)
    pltpu.sync_copy(eids_hbm.at[pl.ds(base, _BPW)], eidx_v)
    bufs = (rows0, rows1)
    sems = (sem0, sem1)
    work = [(qtab_hbm, qidx_v, q_out, c) for c in range(_CH)]
    work += [(etab_hbm, eidx_v, e_out, c) for c in range(_CH)]

    def _start(j, buf, sem):
        tab, idx, _, c = work[j]
        pltpu.async_copy(tab.at[idx.at[pl.ds(c * _CROWS, _CROWS)]], buf, sem)

    _start(0, bufs[0], sems[0])
    for j in range(len(work)):
        if j + 1 < len(work):
            _start(j + 1, bufs[(j + 1) % 2], sems[(j + 1) % 2])
        tab, idx, out, c = work[j]
        buf = bufs[j % 2]
        pltpu.make_async_copy(tab.at[idx.at[pl.ds(c * _CROWS, _CROWS)]],
                              buf, sems[j % 2]).wait()
        pltpu.sync_copy(buf, out.at[pl.ds(base + c * _CROWS, _CROWS)])


@functools.lru_cache(maxsize=None)
def _gather_call():
    return pl.kernel(
        _sc_gather,
        mesh=plsc.VectorSubcoreMesh(core_axis_name="c", subcore_axis_name="s"),
        out_type=[
            jax.ShapeDtypeStruct((_CB, _D), jnp.float32),
            jax.ShapeDtypeStruct((_CB, _D), jnp.float32),
        ],
        scratch_types=[
            pltpu.VMEM((_BPW,), jnp.int32),
            pltpu.VMEM((_BPW,), jnp.int32),
            pltpu.VMEM((_CROWS, _D), jnp.float32),
            pltpu.VMEM((_CROWS, _D), jnp.float32),
            pltpu.SemaphoreType.DMA,
            pltpu.SemaphoreType.DMA,
        ],
    )


_BLK = 1024
_GSTEPS = _CB // _BLK
_C1 = 0.7978845608028654
_C2 = _C1 * 0.044715


def _gelu2(x):
    one = jnp.asarray(1.0, x.dtype)
    c1 = jnp.asarray(_C1, x.dtype)
    c2 = jnp.asarray(_C2, x.dtype)
    t = jnp.tanh(x * (c1 + c2 * (x * x)))
    return x * (one + t)


def _mlp_body(xq_ref, xe_ref, w1_ref, b1_ref, w2_ref, b2_ref,
              oq_ref, oe_ref):
    w1 = w1_ref[...]
    w2 = w2_ref[...]
    b1 = b1_ref[...]
    b2 = b2_ref[...]
    hq = _gelu2(
        jnp.dot(xq_ref[...].astype(jnp.bfloat16), w1,
                preferred_element_type=jnp.float32).astype(jnp.bfloat16)
        + b1)
    oq_ref[...] = jnp.dot(hq, w2, preferred_element_type=jnp.float32) + b2
    he = _gelu2(
        jnp.dot(xe_ref[...].astype(jnp.bfloat16), w1,
                preferred_element_type=jnp.float32).astype(jnp.bfloat16)
        + b1)
    oe_ref[...] = jnp.dot(he, w2, preferred_element_type=jnp.float32) + b2


def _mlp_chunk0(body_args):
    row_in = pl.BlockSpec((_BLK, _D), lambda i: (i, 0))
    row_out = pl.BlockSpec((_BLK, _D), lambda i: (i, 0))
    return pl.pallas_call(
        _mlp_body,
        grid=(_GSTEPS,),
        in_specs=[
            row_in,
            row_in,
            pl.BlockSpec((_D, _H), lambda i: (0, 0)),
            pl.BlockSpec((1, _H), lambda i: (0, 0)),
            pl.BlockSpec((_H, _D), lambda i: (0, 0)),
            pl.BlockSpec((1, _D), lambda i: (0, 0)),
        ],
        out_specs=[row_out, row_out],
        out_shape=[
            jax.ShapeDtypeStruct((_B, _D), jnp.float32),
            jax.ShapeDtypeStruct((_B, _D), jnp.float32),
        ],
    )(*body_args)


def _mlp_chunk1_body(xq_ref, xe_ref, w1_ref, b1_ref, w2_ref, b2_ref,
                     _aq_ref, _ae_ref, oq_ref, oe_ref):
    _mlp_body(xq_ref, xe_ref, w1_ref, b1_ref, w2_ref, b2_ref,
              oq_ref, oe_ref)


def _mlp_chunk1(body_args, oq_prev, oe_prev):
    row_in = pl.BlockSpec((_BLK, _D), lambda i: (i, 0))
    row_out = pl.BlockSpec((_BLK, _D), lambda i: (i + _GSTEPS, 0))
    return pl.pallas_call(
        _mlp_chunk1_body,
        grid=(_GSTEPS,),
        in_specs=[
            row_in,
            row_in,
            pl.BlockSpec((_D, _H), lambda i: (0, 0)),
            pl.BlockSpec((1, _H), lambda i: (0, 0)),
            pl.BlockSpec((_H, _D), lambda i: (0, 0)),
            pl.BlockSpec((1, _D), lambda i: (0, 0)),
            pl.BlockSpec(memory_space=pl.ANY),
            pl.BlockSpec(memory_space=pl.ANY),
        ],
        out_specs=[row_out, row_out],
        out_shape=[
            jax.ShapeDtypeStruct((_B, _D), jnp.float32),
            jax.ShapeDtypeStruct((_B, _D), jnp.float32),
        ],
        input_output_aliases={6: 0, 7: 1},
    )(*body_args, oq_prev, oe_prev)


@jax.jit
def kernel(query_ids, entity_ids, query_emb, ent_emb, W1, b1, W2, b2):
    qids = query_ids.astype(jnp.int32)
    eids = entity_ids.astype(jnp.int32)
    w1 = W1.astype(jnp.bfloat16)
    w2 = (0.5 * W2).astype(jnp.bfloat16)
    b1r = b1.astype(jnp.bfloat16).reshape(1, _H)
    b2r = b2.reshape(1, _D)
    gather = _gather_call()
    qr0, er0 = gather(query_emb, ent_emb, qids[:_CB], eids[:_CB])
    qr1, er1 = gather(query_emb, ent_emb, qids[_CB:], eids[_CB:])
    oq, oe = _mlp_chunk0((qr0, er0, w1, b1r, w2, b2r))
    return _mlp_chunk1((qr1, er1, w1, b1r, w2, b2r), oq, oe)
